# baseline (speedup 1.0000x reference)
import functools

import jax
import jax.numpy as jnp
from jax.experimental import pallas as pl
from jax.experimental.pallas import tpu as pltpu

_POOL_PAD = 128
_NEG = -3e38


def _sim_body(x_ref, pk_ref, out_ref, *, pool, seq, bblk):
    rows = x_ref.shape[0]
    x = x_ref[...]
    j = jax.lax.broadcasted_iota(jnp.int32, (bblk, rows), 1)
    i = jax.lax.broadcasted_iota(jnp.int32, (bblk, rows), 0)
    sel = jnp.where(j // seq == i, jnp.float32(1.0 / seq), jnp.float32(0.0))
    xm = jax.lax.dot_general(sel, x, (((1,), (0,)), ((), ())),
                             preferred_element_type=jnp.float32)
    ss = jnp.sum(xm * xm, axis=1, keepdims=True)
    xn = xm * jax.lax.rsqrt(jnp.maximum(ss, 1e-12))
    pk = pk_ref[...]
    ps = jnp.sum(pk * pk, axis=1, keepdims=True)
    pn = pk * jax.lax.rsqrt(jnp.maximum(ps, 1e-12))
    sim = jax.lax.dot_general(xn, pn, (((1,), (1,)), ((), ())),
                              preferred_element_type=jnp.float32)
    out_ref[:, :pool] = sim
    out_ref[:, pool:] = jnp.full((bblk, _POOL_PAD - pool), _NEG, jnp.float32)


def _topk_hist_body(sim_ref, out_ref, *, top_k):
    sim = sim_ref[...]
    b = sim.shape[0]
    col = jax.lax.broadcasted_iota(jnp.int32, (b, _POOL_PAD), 1)
    hist2d = jnp.zeros((b, _POOL_PAD), jnp.int32)
    work = sim
    for _ in range(top_k):
        m = jnp.max(work, axis=1, keepdims=True)
        cand = jnp.where(work == m, col, jnp.int32(1 << 30))
        a = jnp.min(cand, axis=1, keepdims=True)
        pick = col == a
        hist2d = hist2d + pick.astype(jnp.int32)
        work = jnp.where(pick, _NEG, work)
    hist = jnp.sum(hist2d, axis=0, keepdims=True)
    colr = jax.lax.broadcasted_iota(jnp.int32, (1, _POOL_PAD), 1)
    key = hist * 256 + (255 - colr)
    for t in range(top_k):
        m = jnp.max(key)
        out_ref[t] = 255 - (m % 256)
        key = jnp.where(key == m, jnp.int32(-1), key)


def _gather_body(ids_ref, prompt_ref, out_ref, *, top_k, length):
    blk = out_ref.shape[0]
    for t in range(top_k):
        row = prompt_ref[pl.ds(ids_ref[t], 1)]
        out_ref[:, t * length:(t + 1) * length, :] = jnp.broadcast_to(
            row, (blk, length, row.shape[2]))


def kernel(x_embed, prompt, prompt_key):
    b, s, d = x_embed.shape
    pool, length, _ = prompt.shape
    top_k = 8

    bblk = 16
    x2 = x_embed.reshape(b * s, d)
    sim = pl.pallas_call(
        functools.partial(_sim_body, pool=pool, seq=s, bblk=bblk),
        grid=(b // bblk,),
        in_specs=[
            pl.BlockSpec((bblk * s, d), lambda i: (i, 0)),
            pl.BlockSpec((pool, d), lambda i: (0, 0)),
        ],
        out_specs=pl.BlockSpec((bblk, _POOL_PAD), lambda i: (i, 0)),
        out_shape=jax.ShapeDtypeStruct((b, _POOL_PAD), jnp.float32),
    )(x2, prompt_key)

    ids = pl.pallas_call(
        functools.partial(_topk_hist_body, top_k=top_k),
        in_specs=[pl.BlockSpec((b, _POOL_PAD), lambda: (0, 0))],
        out_specs=pl.BlockSpec(memory_space=pltpu.SMEM),
        out_shape=jax.ShapeDtypeStruct((top_k,), jnp.int32),
    )(sim)

    gblk = 32
    out = pl.pallas_call(
        functools.partial(_gather_body, top_k=top_k, length=length),
        grid=(b // gblk,),
        in_specs=[
            pl.BlockSpec(memory_space=pltpu.SMEM),
            pl.BlockSpec((pool, length, d), lambda i: (0, 0, 0)),
        ],
        out_specs=pl.BlockSpec((gblk, top_k * length, d), lambda i: (i, 0, 0)),
        out_shape=jax.ShapeDtypeStruct((b, top_k * length, d), jnp.float32),
    )(ids, prompt)
    return out


# speedup vs baseline: 1.6924x; 1.6924x over previous
import functools

import jax
import jax.numpy as jnp
from jax.experimental import pallas as pl
from jax.experimental.pallas import tpu as pltpu

_POOL_PAD = 128
_NEG = -3e38


def _seqsum_body(x_ref, out_ref, *, seq, sblk):
    pid = pl.program_id(0)
    x = x_ref[...]
    spos = jax.lax.broadcasted_iota(jnp.int32, (1, sblk, 1), 1) + pid * sblk
    xv = jnp.where(spos < seq, x, jnp.float32(0.0))
    part = jnp.sum(xv, axis=1)

    @pl.when(pid == 0)
    def _init():
        out_ref[...] = jnp.zeros_like(out_ref)

    out_ref[...] += part


def _sim_topk_body(xsum_ref, pk_ref, out_ref, *, pool, seq, top_k):
    xm = xsum_ref[...] * jnp.float32(1.0 / seq)
    b = xm.shape[0]
    ss = jnp.sum(xm * xm, axis=1, keepdims=True)
    xn = xm * jax.lax.rsqrt(jnp.maximum(ss, 1e-12))
    pk = pk_ref[...]
    ps = jnp.sum(pk * pk, axis=1, keepdims=True)
    pn = pk * jax.lax.rsqrt(jnp.maximum(ps, 1e-12))
    sim = jax.lax.dot_general(xn, pn, (((1,), (1,)), ((), ())),
                              preferred_element_type=jnp.float32)
    work = jnp.concatenate(
        [sim, jnp.full((b, _POOL_PAD - pool), _NEG, jnp.float32)], axis=1)
    col = jax.lax.broadcasted_iota(jnp.int32, (b, _POOL_PAD), 1)
    hist2d = jnp.zeros((b, _POOL_PAD), jnp.int32)
    for _ in range(top_k):
        m = jnp.max(work, axis=1, keepdims=True)
        cand = jnp.where(work == m, col, jnp.int32(1 << 30))
        a = jnp.min(cand, axis=1, keepdims=True)
        pick = col == a
        hist2d = hist2d + pick.astype(jnp.int32)
        work = jnp.where(pick, _NEG, work)
    hist = jnp.sum(hist2d, axis=0, keepdims=True)
    colr = jax.lax.broadcasted_iota(jnp.int32, (1, _POOL_PAD), 1)
    key = hist * 256 + (255 - colr)
    for t in range(top_k):
        m = jnp.max(key)
        out_ref[t] = 255 - (m % 256)
        key = jnp.where(key == m, jnp.int32(-1), key)


def _gather_body(ids_ref, prompt_ref, out_ref, *, top_k, length):
    blk = out_ref.shape[0]
    for t in range(top_k):
        row = prompt_ref[pl.ds(ids_ref[t], 1)]
        out_ref[:, t * length:(t + 1) * length, :] = jnp.broadcast_to(
            row, (blk, length, row.shape[2]))


def kernel(x_embed, prompt, prompt_key):
    b, s, d = x_embed.shape
    pool, length, _ = prompt.shape
    top_k = 8

    sblk = 8
    s_steps = (s + sblk - 1) // sblk
    xsum = pl.pallas_call(
        functools.partial(_seqsum_body, seq=s, sblk=sblk),
        grid=(s_steps,),
        in_specs=[pl.BlockSpec((b, sblk, d), lambda i: (0, i, 0))],
        out_specs=pl.BlockSpec((b, d), lambda i: (0, 0)),
        out_shape=jax.ShapeDtypeStruct((b, d), jnp.float32),
    )(x_embed)

    ids = pl.pallas_call(
        functools.partial(_sim_topk_body, pool=pool, seq=s, top_k=top_k),
        in_specs=[
            pl.BlockSpec((b, d), lambda: (0, 0)),
            pl.BlockSpec((pool, d), lambda: (0, 0)),
        ],
        out_specs=pl.BlockSpec(memory_space=pltpu.SMEM),
        out_shape=jax.ShapeDtypeStruct((top_k,), jnp.int32),
    )(xsum, prompt_key)

    gblk = 32
    out = pl.pallas_call(
        functools.partial(_gather_body, top_k=top_k, length=length),
        grid=(b // gblk,),
        in_specs=[
            pl.BlockSpec(memory_space=pltpu.SMEM),
            pl.BlockSpec((pool, length, d), lambda i: (0, 0, 0)),
        ],
        out_specs=pl.BlockSpec((gblk, top_k * length, d), lambda i: (i, 0, 0)),
        out_shape=jax.ShapeDtypeStruct((b, top_k * length, d), jnp.float32),
    )(ids, prompt)
    return out


# speedup vs baseline: 1.7429x; 1.0299x over previous
import functools

import jax
import jax.numpy as jnp
from jax.experimental import pallas as pl
from jax.experimental.pallas import tpu as pltpu

_POOL_PAD = 128
_NEG = -3e38


def _seqsum_body(*refs, seq, sblk, nsplit):
    x_refs, out_refs = refs[:nsplit], refs[nsplit:]
    pid = pl.program_id(0)
    spos = jax.lax.broadcasted_iota(jnp.int32, (1, sblk, 1), 1) + pid * sblk
    valid = spos < seq
    for x_ref, out_ref in zip(x_refs, out_refs):
        x = x_ref[...]
        xv = jnp.where(valid, x, jnp.float32(0.0))
        part = jnp.sum(xv, axis=1)

        @pl.when(pid == 0)
        def _init(out_ref=out_ref, part=part):
            out_ref[...] = part

        @pl.when(pid > 0)
        def _acc(out_ref=out_ref, part=part):
            out_ref[...] += part


def _sim_topk_body(xsum_ref, pk_ref, out_ref, *, pool, seq, top_k):
    xm = xsum_ref[...] * jnp.float32(1.0 / seq)
    b = xm.shape[0]
    ss = jnp.sum(xm * xm, axis=1, keepdims=True)
    xn = xm * jax.lax.rsqrt(jnp.maximum(ss, 1e-12))
    pk = pk_ref[...]
    ps = jnp.sum(pk * pk, axis=1, keepdims=True)
    pn = pk * jax.lax.rsqrt(jnp.maximum(ps, 1e-12))
    sim = jax.lax.dot_general(xn, pn, (((1,), (1,)), ((), ())),
                              preferred_element_type=jnp.float32)
    work = jnp.concatenate(
        [sim, jnp.full((b, _POOL_PAD - pool), _NEG, jnp.float32)], axis=1)
    col = jax.lax.broadcasted_iota(jnp.int32, (b, _POOL_PAD), 1)
    hist2d = jnp.zeros((b, _POOL_PAD), jnp.int32)
    for _ in range(top_k):
        m = jnp.max(work, axis=1, keepdims=True)
        cand = jnp.where(work == m, col, jnp.int32(1 << 30))
        a = jnp.min(cand, axis=1, keepdims=True)
        pick = col == a
        hist2d = hist2d + pick.astype(jnp.int32)
        work = jnp.where(pick, _NEG, work)
    hist = jnp.sum(hist2d, axis=0, keepdims=True)
    colr = jax.lax.broadcasted_iota(jnp.int32, (1, _POOL_PAD), 1)
    key = hist * 256 + (255 - colr)
    for t in range(top_k):
        m = jnp.max(key)
        out_ref[t] = 255 - (m % 256)
        key = jnp.where(key == m, jnp.int32(-1), key)


def _gather_body(ids_ref, prompt_ref, out_ref, *, top_k, length):
    blk = out_ref.shape[0]
    for t in range(top_k):
        row = prompt_ref[pl.ds(ids_ref[t], 1)]
        out_ref[:, t * length:(t + 1) * length, :] = jnp.broadcast_to(
            row, (blk, length, row.shape[2]))


def kernel(x_embed, prompt, prompt_key):
    b, s, d = x_embed.shape
    pool, length, _ = prompt.shape
    top_k = 8

    sblk = 8
    nsplit = 4
    bsub = b // nsplit
    s_steps = (s + sblk - 1) // sblk
    xparts = pl.pallas_call(
        functools.partial(_seqsum_body, seq=s, sblk=sblk, nsplit=nsplit),
        grid=(s_steps,),
        in_specs=[
            pl.BlockSpec((bsub, sblk, d),
                         functools.partial(lambda t, i: (t, i, 0), t))
            for t in range(nsplit)
        ],
        out_specs=[pl.BlockSpec((bsub, d), lambda i: (0, 0))
                   for _ in range(nsplit)],
        out_shape=[jax.ShapeDtypeStruct((bsub, d), jnp.float32)
                   for _ in range(nsplit)],
    )(*([x_embed] * nsplit))
    xsum = jnp.concatenate(xparts, axis=0)

    ids = pl.pallas_call(
        functools.partial(_sim_topk_body, pool=pool, seq=s, top_k=top_k),
        in_specs=[
            pl.BlockSpec((b, d), lambda: (0, 0)),
            pl.BlockSpec((pool, d), lambda: (0, 0)),
        ],
        out_specs=pl.BlockSpec(memory_space=pltpu.SMEM),
        out_shape=jax.ShapeDtypeStruct((top_k,), jnp.int32),
    )(xsum, prompt_key)

    gblk = 32
    out = pl.pallas_call(
        functools.partial(_gather_body, top_k=top_k, length=length),
        grid=(b // gblk,),
        in_specs=[
            pl.BlockSpec(memory_space=pltpu.SMEM),
            pl.BlockSpec((pool, length, d), lambda i: (0, 0, 0)),
        ],
        out_specs=pl.BlockSpec((gblk, top_k * length, d), lambda i: (i, 0, 0)),
        out_shape=jax.ShapeDtypeStruct((b, top_k * length, d), jnp.float32),
    )(ids, prompt)
    return out


# speedup vs baseline: 1.8444x; 1.0583x over previous
import functools

import jax
import jax.numpy as jnp
from jax.experimental import pallas as pl
from jax.experimental.pallas import tpu as pltpu

_POOL_PAD = 128
_NEG = -3e38
_NBUF = 4
_CHUNKS = 16


def _ids_body(x_hbm, pk_ref, out_ref, xsum_ref, *bufs_and_sems,
              pool, seq, top_k):
    bufs, sems = bufs_and_sems[:_NBUF], bufs_and_sems[_NBUF:]
    cb = bufs[0].shape[0]

    def start(c):
        pltpu.make_async_copy(
            x_hbm.at[pl.ds(c * cb, cb)], bufs[c % _NBUF],
            sems[c % _NBUF]).start()

    def wait(c):
        pltpu.make_async_copy(
            x_hbm.at[pl.ds(c * cb, cb)], bufs[c % _NBUF],
            sems[c % _NBUF]).wait()

    for c in range(_NBUF):
        start(c)
    for c in range(_CHUNKS):
        wait(c)
        xsum_ref[c * cb:(c + 1) * cb, :] = jnp.sum(bufs[c % _NBUF][...],
                                                   axis=1)
        if c + _NBUF < _CHUNKS:
            start(c + _NBUF)

    xm = xsum_ref[...] * jnp.float32(1.0 / seq)
    b = xm.shape[0]
    ss = jnp.sum(xm * xm, axis=1, keepdims=True)
    xn = xm * jax.lax.rsqrt(jnp.maximum(ss, 1e-12))
    pk = pk_ref[...]
    ps = jnp.sum(pk * pk, axis=1, keepdims=True)
    pn = pk * jax.lax.rsqrt(jnp.maximum(ps, 1e-12))
    sim = jax.lax.dot_general(xn, pn, (((1,), (1,)), ((), ())),
                              preferred_element_type=jnp.float32)
    work = jnp.concatenate(
        [sim, jnp.full((b, _POOL_PAD - pool), _NEG, jnp.float32)], axis=1)
    col = jax.lax.broadcasted_iota(jnp.int32, (b, _POOL_PAD), 1)
    hist2d = jnp.zeros((b, _POOL_PAD), jnp.int32)
    for _ in range(top_k):
        m = jnp.max(work, axis=1, keepdims=True)
        cand = jnp.where(work == m, col, jnp.int32(1 << 30))
        a = jnp.min(cand, axis=1, keepdims=True)
        pick = col == a
        hist2d = hist2d + pick.astype(jnp.int32)
        work = jnp.where(pick, _NEG, work)
    hist = jnp.sum(hist2d, axis=0, keepdims=True)
    colr = jax.lax.broadcasted_iota(jnp.int32, (1, _POOL_PAD), 1)
    key = hist * 256 + (255 - colr)
    for t in range(top_k):
        m = jnp.max(key)
        out_ref[t] = 255 - (m % 256)
        key = jnp.where(key == m, jnp.int32(-1), key)


def _gather_body(ids_ref, prompt_ref, out_ref, *, top_k, length):
    blk = out_ref.shape[0]
    for t in range(top_k):
        row = prompt_ref[pl.ds(ids_ref[t], 1)]
        out_ref[:, t * length:(t + 1) * length, :] = jnp.broadcast_to(
            row, (blk, length, row.shape[2]))


def kernel(x_embed, prompt, prompt_key):
    b, s, d = x_embed.shape
    pool, length, _ = prompt.shape
    top_k = 8
    cb = b // _CHUNKS

    ids = pl.pallas_call(
        functools.partial(_ids_body, pool=pool, seq=s, top_k=top_k),
        in_specs=[
            pl.BlockSpec(memory_space=pltpu.HBM),
            pl.BlockSpec((pool, d), lambda: (0, 0)),
        ],
        out_specs=pl.BlockSpec(memory_space=pltpu.SMEM),
        out_shape=jax.ShapeDtypeStruct((top_k,), jnp.int32),
        scratch_shapes=(
            [pltpu.VMEM((b, d), jnp.float32)]
            + [pltpu.VMEM((cb, s, d), jnp.float32) for _ in range(_NBUF)]
            + [pltpu.SemaphoreType.DMA for _ in range(_NBUF)]
        ),
    )(x_embed, prompt_key)

    gblk = 32
    out = pl.pallas_call(
        functools.partial(_gather_body, top_k=top_k, length=length),
        grid=(b // gblk,),
        in_specs=[
            pl.BlockSpec(memory_space=pltpu.SMEM),
            pl.BlockSpec((pool, length, d), lambda i: (0, 0, 0)),
        ],
        out_specs=pl.BlockSpec((gblk, top_k * length, d), lambda i: (i, 0, 0)),
        out_shape=jax.ShapeDtypeStruct((b, top_k * length, d), jnp.float32),
    )(ids, prompt)
    return out
